# baseline (speedup 1.0000x reference)
import functools

import jax
import jax.numpy as jnp
from jax import lax
from jax.experimental import pallas as pl
from jax.experimental.pallas import tpu as pltpu
from jax.experimental.pallas import tpu_sc as plsc

N = 100000
E = 1600000
HD = 32
G = 64
NB = 4000
EB = 16000
NCORES = 2
NSUB = 16
LANES = 16


def _lin_body(x_ref, w_ref, b_ref, o_ref):
    o_ref[...] = (
        jnp.dot(x_ref[...], w_ref[...], preferred_element_type=jnp.float32)
        + b_ref[...]
    )


def _linear(x, w, b, blk):
    m, kdim = x.shape
    n = w.shape[1]
    return pl.pallas_call(
        _lin_body,
        grid=(m // blk,),
        in_specs=[
            pl.BlockSpec((blk, kdim), lambda i: (i, 0)),
            pl.BlockSpec((kdim, n), lambda i: (0, 0)),
            pl.BlockSpec((1, n), lambda i: (0, 0)),
        ],
        out_specs=pl.BlockSpec((blk, n), lambda i: (i, 0)),
        out_shape=jax.ShapeDtypeStruct((m, n), jnp.float32),
    )(x, w, b.reshape(1, n))


def _mlp_body(h_ref, a_ref, w1_ref, b1_ref, w2_ref, b2_ref, z_ref, s_ref):
    z = h_ref[...] + a_ref[...]
    z = jnp.maximum(
        jnp.dot(z, w1_ref[...], preferred_element_type=jnp.float32) + b1_ref[...],
        0.0,
    )
    z = jnp.dot(z, w2_ref[...], preferred_element_type=jnp.float32) + b2_ref[...]
    z_ref[...] = z
    s = jnp.sum(z, axis=0, keepdims=True)
    q = jnp.sum(z * z, axis=0, keepdims=True)
    upd = jnp.concatenate([s, q, jnp.zeros((6, HD), jnp.float32)], axis=0)
    prev = jnp.where(pl.program_id(0) == 0, jnp.zeros_like(upd), s_ref[...])
    s_ref[...] = prev + upd


def _mlp_stats(h, agg, w1, b1, w2, b2):
    return pl.pallas_call(
        _mlp_body,
        grid=(N // NB,),
        in_specs=[
            pl.BlockSpec((NB, HD), lambda i: (i, 0)),
            pl.BlockSpec((NB, HD), lambda i: (i, 0)),
            pl.BlockSpec((HD, HD), lambda i: (0, 0)),
            pl.BlockSpec((1, HD), lambda i: (0, 0)),
            pl.BlockSpec((HD, HD), lambda i: (0, 0)),
            pl.BlockSpec((1, HD), lambda i: (0, 0)),
        ],
        out_specs=[
            pl.BlockSpec((NB, HD), lambda i: (i, 0)),
            pl.BlockSpec((8, HD), lambda i: (0, 0)),
        ],
        out_shape=[
            jax.ShapeDtypeStruct((N, HD), jnp.float32),
            jax.ShapeDtypeStruct((8, HD), jnp.float32),
        ],
    )(h, agg, w1, b1.reshape(1, HD), w2, b2.reshape(1, HD))


def _norm_body(z_ref, s_ref, g_ref, b_ref, o_ref):
    mu = s_ref[0:1, :] * (1.0 / N)
    var = s_ref[1:2, :] * (1.0 / N) - mu * mu
    scale = g_ref[...] * lax.rsqrt(var + 1e-5)
    o_ref[...] = jnp.maximum((z_ref[...] - mu) * scale + b_ref[...], 0.0)


def _norm_relu(z, stats, g, b):
    return pl.pallas_call(
        _norm_body,
        grid=(N // NB,),
        in_specs=[
            pl.BlockSpec((NB, HD), lambda i: (i, 0)),
            pl.BlockSpec((8, HD), lambda i: (0, 0)),
            pl.BlockSpec((1, HD), lambda i: (0, 0)),
            pl.BlockSpec((1, HD), lambda i: (0, 0)),
        ],
        out_specs=pl.BlockSpec((NB, HD), lambda i: (i, 0)),
        out_shape=jax.ShapeDtypeStruct((N, HD), jnp.float32),
    )(z, stats, g.reshape(1, HD), b.reshape(1, HD))


def _tail_body(z_ref, s_ref, g_ref, b_ref, wf_ref, bf_ref, bat_ref, wo_ref,
               bo_ref, o_ref, acc_ref):
    i = pl.program_id(0)
    mu = s_ref[0:1, :] * (1.0 / N)
    var = s_ref[1:2, :] * (1.0 / N) - mu * mu
    scale = g_ref[...] * lax.rsqrt(var + 1e-5)
    h2 = jnp.maximum((z_ref[...] - mu) * scale + b_ref[...], 0.0)
    hf = jnp.maximum(
        jnp.dot(h2, wf_ref[...], preferred_element_type=jnp.float32) + bf_ref[...],
        0.0,
    )
    hf_ext = jnp.concatenate([hf, jnp.ones((NB, 1), jnp.float32)], axis=1)
    seg = lax.broadcasted_iota(jnp.int32, (NB, G), 1)
    onehot = (seg == bat_ref[...]).astype(jnp.float32)
    upd = lax.dot_general(onehot, hf_ext, (((0,), (0,)), ((), ())),
                          preferred_element_type=jnp.float32)

    @pl.when(i == 0)
    def _():
        acc_ref[...] = jnp.zeros_like(acc_ref)

    acc_ref[...] += upd

    @pl.when(i == N // NB - 1)
    def _():
        acc = acc_ref[...]
        pooled = acc[:, 0:HD] / jnp.maximum(acc[:, HD:HD + 1], 1.0)
        o_ref[...] = (
            jnp.dot(pooled, wo_ref[...], preferred_element_type=jnp.float32)
            + bo_ref[...]
        )


def _tail(z, stats, g, b, w_fin, b_fin, batch_row, w_out, b_out):
    c = w_out.shape[1]
    return pl.pallas_call(
        _tail_body,
        grid=(N // NB,),
        in_specs=[
            pl.BlockSpec((NB, HD), lambda i: (i, 0)),
            pl.BlockSpec((8, HD), lambda i: (0, 0)),
            pl.BlockSpec((1, HD), lambda i: (0, 0)),
            pl.BlockSpec((1, HD), lambda i: (0, 0)),
            pl.BlockSpec((HD, HD), lambda i: (0, 0)),
            pl.BlockSpec((1, HD), lambda i: (0, 0)),
            pl.BlockSpec((NB, 1), lambda i: (i, 0)),
            pl.BlockSpec((HD, c), lambda i: (0, 0)),
            pl.BlockSpec((1, c), lambda i: (0, 0)),
        ],
        out_specs=pl.BlockSpec((G, c), lambda i: (0, 0)),
        out_shape=jax.ShapeDtypeStruct((G, c), jnp.float32),
        scratch_shapes=[pltpu.VMEM((G, HD + 1), jnp.float32)],
    )(z, stats, g.reshape(1, HD), b.reshape(1, HD), w_fin,
      b_fin.reshape(1, HD), batch_row, w_out, b_out.reshape(1, c))


def _build_sc_msg(n, e, interpret=False):
    n_pad = -(-n // (NCORES * NSUB * 8)) * (NCORES * NSUB * 8)
    npc = n_pad // NCORES
    dummy = 512
    agg_rows = npc + dummy
    kc = 128
    nchunks = e // kc
    rows_per_tile = npc // NSUB
    full = nchunks // NSUB
    extra = nchunks - full * NSUB
    zrows = 64
    spans = []
    o = 0
    while o < rows_per_tile:
        sz = min(zrows, rows_per_tile - o)
        spans.append((o, sz))
        o += sz

    mesh = plsc.VectorSubcoreMesh(core_axis_name="c", subcore_axis_name="s",
                                  num_cores=NCORES, num_subcores=NSUB)

    @functools.partial(
        pl.kernel,
        out_type=jax.ShapeDtypeStruct((n_pad, HD), jnp.float32),
        mesh=mesh,
        scratch_types=[
            pltpu.VMEM_SHARED((agg_rows, HD), jnp.float32),
            [pltpu.VMEM((kc,), jnp.int32) for _ in range(3)],
            [pltpu.VMEM((kc,), jnp.int32) for _ in range(3)],
            [pltpu.VMEM((kc, HD), jnp.float32) for _ in range(3)],
            [pltpu.VMEM((kc * HD,), jnp.float32) for _ in range(3)],
            pltpu.VMEM((zrows, HD), jnp.float32),
            [pltpu.SemaphoreType.DMA for _ in range(3)],
            [pltpu.SemaphoreType.DMA for _ in range(3)],
            [pltpu.SemaphoreType.DMA for _ in range(3)],
        ],
        compiler_params=pltpu.CompilerParams(use_tc_tiling_on_sc=False),
        interpret=interpret,
    )
    def sc_msg(h_hbm, ea_hbm, src_hbm, dst_hbm, out_hbm,
               agg_sh, src_v, dst_v, rows_v, ea_v, zb,
               sem_ld, sem_g, sem_sc):
        c = lax.axis_index("c")
        s = lax.axis_index("s")
        base = c * npc
        row0 = s * rows_per_tile

        @plsc.parallel_loop(0, zrows, unroll=8)
        def _zz(j):
            zb[j, pl.ds(0, LANES)] = jnp.zeros((LANES,), jnp.float32)
            zb[j, pl.ds(LANES, LANES)] = jnp.zeros((LANES,), jnp.float32)

        for ofs, sz in spans:
            pltpu.sync_copy(zb.at[pl.ds(0, sz)],
                            agg_sh.at[pl.ds(row0 + ofs, sz)])
        plsc.subcore_barrier()

        nq = full + jnp.where(s < extra, 1, 0)

        def issue_ld(q, b):
            off = (q * NSUB + s) * kc
            pltpu.async_copy(src_hbm.at[pl.ds(off, kc)], src_v[b], sem_ld[b])
            pltpu.async_copy(dst_hbm.at[pl.ds(off, kc)], dst_v[b], sem_ld[b])
            pltpu.async_copy(ea_hbm.at[pl.ds(off * HD, kc * HD)], ea_v[b],
                             sem_ld[b])

        def wait_ld(b):
            pltpu.make_async_copy(src_hbm.at[pl.ds(0, kc)], src_v[b],
                                  sem_ld[b]).wait()
            pltpu.make_async_copy(dst_hbm.at[pl.ds(0, kc)], dst_v[b],
                                  sem_ld[b]).wait()
            pltpu.make_async_copy(ea_hbm.at[pl.ds(0, kc * HD)], ea_v[b],
                                  sem_ld[b]).wait()

        def issue_gather(b):
            pltpu.async_copy(h_hbm.at[src_v[b]], rows_v[b], sem_g[b])

        def wait_gather(b):
            pltpu.make_async_copy(h_hbm.at[src_v[b]], rows_v[b],
                                  sem_g[b]).wait()

        def issue_scatter(b):
            pltpu.async_copy(rows_v[b], agg_sh.at[dst_v[b]], sem_sc[b],
                             add=True)

        def wait_scatter(b):
            pltpu.make_async_copy(rows_v[b], agg_sh.at[dst_v[b]],
                                  sem_sc[b]).wait()

        @pl.when(0 < nq)
        def _():
            issue_ld(0, 0)

        @pl.when(1 < nq)
        def _():
            issue_ld(1, 1)

        @pl.when(0 < nq)
        def _():
            wait_ld(0)
            issue_gather(0)

        @pl.loop(0, (nq + 2) // 3)
        def _macro(m):
            for j in range(3):
                q = m * 3 + j
                jn1 = (j + 1) % 3
                jn2 = (j + 2) % 3

                @pl.when(q + 2 < nq)
                def _(q=q, jn2=jn2):
                    @pl.when(q >= 1)
                    def _():
                        wait_scatter(jn2)

                    issue_ld(q + 2, jn2)

                @pl.when(q + 1 < nq)
                def _(q=q, jn1=jn1):
                    wait_ld(jn1)
                    issue_gather(jn1)

                @pl.when(q < nq)
                def _(q=q, j=j):
                    wait_gather(j)

                    @plsc.parallel_loop(0, kc, unroll=4)
                    def _rows(r):
                        a0 = rows_v[j][r, pl.ds(0, LANES)] + ea_v[j][pl.ds(r * HD, LANES)]
                        rows_v[j][r, pl.ds(0, LANES)] = jnp.maximum(a0, 0.0)
                        a1 = rows_v[j][r, pl.ds(LANES, LANES)] + ea_v[j][pl.ds(r * HD + LANES, LANES)]
                        rows_v[j][r, pl.ds(LANES, LANES)] = jnp.maximum(a1, 0.0)

                    @plsc.parallel_loop(0, kc // LANES, unroll=2)
                    def _didx(t):
                        d = dst_v[j][pl.ds(t * LANES, LANES)]
                        loc = d - base
                        oob = (loc < 0) | (loc >= npc)
                        dl = jnp.where(oob, npc + (d & (dummy - 1)), loc)
                        dst_v[j][pl.ds(t * LANES, LANES)] = dl

                    issue_scatter(j)

        for j in range(3):
            @pl.when(nq > j)
            def _(j=j):
                wait_scatter(j)

        plsc.subcore_barrier()

        for ofs, sz in spans:
            pltpu.sync_copy(agg_sh.at[pl.ds(row0 + ofs, sz)],
                            zb.at[pl.ds(0, sz)])
            pltpu.sync_copy(zb.at[pl.ds(0, sz)],
                            out_hbm.at[pl.ds(base + row0 + ofs, sz)])

    return sc_msg


@functools.lru_cache(maxsize=None)
def _sc_msg_cached():
    return _build_sc_msg(N, E)


def _sc_msg(h, ea, src, dst):
    return _sc_msg_cached()(h, ea, src, dst)[:N]


def kernel(x, edge_index, edge_attr, batch, w_node, b_node, w_edge, b_edge,
           w1_0, b1_0, w2_0, b2_0, bn_g_0, bn_b_0,
           w1_1, b1_1, w2_1, b2_1, bn_g_1, bn_b_1,
           w_fin, b_fin, w_out, b_out):
    src = edge_index[0]
    dst = edge_index[1]
    h0 = _linear(x, w_node, b_node, NB)
    ea = _linear(edge_attr, w_edge, b_edge, EB).reshape(E * HD)
    agg0 = _sc_msg(h0, ea, src, dst)
    z0, s0 = _mlp_stats(h0, agg0, w1_0, b1_0, w2_0, b2_0)
    h1 = _norm_relu(z0, s0, bn_g_0, bn_b_0)
    agg1 = _sc_msg(h1, ea, src, dst)
    z1, s1 = _mlp_stats(h1, agg1, w1_1, b1_1, w2_1, b2_1)
    return _tail(z1, s1, bn_g_1, bn_b_1, w_fin, b_fin,
                 batch.reshape(N, 1), w_out, b_out)


# speedup vs baseline: 1.2911x; 1.2911x over previous
import functools

import jax
import jax.numpy as jnp
import numpy as np
from jax import lax
from jax.experimental import pallas as pl
from jax.experimental.pallas import tpu as pltpu
from jax.experimental.pallas import tpu_sc as plsc

N = 100000
E = 1600000
DE = 16
HD = 32
G = 64
NB = 4000
EB = 16000
RB8 = 2000
NCORES = 2
NSUB = 16
LANES = 16

_pm = np.empty(128, np.int32)
for _m in range(64):
    _pm[_m] = 8 * (_m // 4) + _m % 4
    _pm[64 + _m] = 8 * (_m // 4) + 4 + _m % 4
PERM128 = _pm


def _lin_body(x_ref, w_ref, b_ref, o_ref):
    o_ref[...] = (
        jnp.dot(x_ref[...], w_ref[...], preferred_element_type=jnp.float32)
        + b_ref[...]
    )


def _linear(x, w, b, blk):
    m, kdim = x.shape
    n = w.shape[1]
    return pl.pallas_call(
        _lin_body,
        grid=(m // blk,),
        in_specs=[
            pl.BlockSpec((blk, kdim), lambda i: (i, 0)),
            pl.BlockSpec((kdim, n), lambda i: (0, 0)),
            pl.BlockSpec((1, n), lambda i: (0, 0)),
        ],
        out_specs=pl.BlockSpec((blk, n), lambda i: (i, 0)),
        out_shape=jax.ShapeDtypeStruct((m, n), jnp.float32),
    )(x, w, b.reshape(1, n))


def _edge2_body(a_ref, wa_ref, wb_ref, b_ref, oa_ref, ob_ref):
    x = a_ref[...]
    oa_ref[...] = (
        jnp.dot(x, wa_ref[...], preferred_element_type=jnp.float32) + b_ref[...]
    )
    ob_ref[...] = (
        jnp.dot(x, wb_ref[...], preferred_element_type=jnp.float32) + b_ref[...]
    )


def _edge_linear_packed(af, w8a, w8b, b128):
    rows = af.shape[0]
    return pl.pallas_call(
        _edge2_body,
        grid=(rows // RB8,),
        in_specs=[
            pl.BlockSpec((RB8, 128), lambda i: (i, 0)),
            pl.BlockSpec((128, 128), lambda i: (0, 0)),
            pl.BlockSpec((128, 128), lambda i: (0, 0)),
            pl.BlockSpec((1, 128), lambda i: (0, 0)),
        ],
        out_specs=[
            pl.BlockSpec((RB8, 128), lambda i: (i, 0)),
            pl.BlockSpec((RB8, 128), lambda i: (i, 0)),
        ],
        out_shape=[
            jax.ShapeDtypeStruct((rows, 128), jnp.float32),
            jax.ShapeDtypeStruct((rows, 128), jnp.float32),
        ],
    )(af, w8a, w8b, b128.reshape(1, 128))


def _mlp_body(h_ref, a_ref, w1_ref, b1_ref, w2_ref, b2_ref, z_ref, s_ref):
    z = h_ref[...] + a_ref[...]
    z = jnp.maximum(
        jnp.dot(z, w1_ref[...], preferred_element_type=jnp.float32) + b1_ref[...],
        0.0,
    )
    z = jnp.dot(z, w2_ref[...], preferred_element_type=jnp.float32) + b2_ref[...]
    z_ref[...] = z
    s = jnp.sum(z, axis=0, keepdims=True)
    q = jnp.sum(z * z, axis=0, keepdims=True)
    upd = jnp.concatenate([s, q, jnp.zeros((6, HD), jnp.float32)], axis=0)
    prev = jnp.where(pl.program_id(0) == 0, jnp.zeros_like(upd), s_ref[...])
    s_ref[...] = prev + upd


def _mlp_stats(h, agg, w1, b1, w2, b2):
    return pl.pallas_call(
        _mlp_body,
        grid=(N // NB,),
        in_specs=[
            pl.BlockSpec((NB, HD), lambda i: (i, 0)),
            pl.BlockSpec((NB, HD), lambda i: (i, 0)),
            pl.BlockSpec((HD, HD), lambda i: (0, 0)),
            pl.BlockSpec((1, HD), lambda i: (0, 0)),
            pl.BlockSpec((HD, HD), lambda i: (0, 0)),
            pl.BlockSpec((1, HD), lambda i: (0, 0)),
        ],
        out_specs=[
            pl.BlockSpec((NB, HD), lambda i: (i, 0)),
            pl.BlockSpec((8, HD), lambda i: (0, 0)),
        ],
        out_shape=[
            jax.ShapeDtypeStruct((N, HD), jnp.float32),
            jax.ShapeDtypeStruct((8, HD), jnp.float32),
        ],
    )(h, agg, w1, b1.reshape(1, HD), w2, b2.reshape(1, HD))


def _norm_body(z_ref, s_ref, g_ref, b_ref, o_ref):
    mu = s_ref[0:1, :] * (1.0 / N)
    var = s_ref[1:2, :] * (1.0 / N) - mu * mu
    scale = g_ref[...] * lax.rsqrt(var + 1e-5)
    o_ref[...] = jnp.maximum((z_ref[...] - mu) * scale + b_ref[...], 0.0)


def _norm_relu(z, stats, g, b):
    return pl.pallas_call(
        _norm_body,
        grid=(N // NB,),
        in_specs=[
            pl.BlockSpec((NB, HD), lambda i: (i, 0)),
            pl.BlockSpec((8, HD), lambda i: (0, 0)),
            pl.BlockSpec((1, HD), lambda i: (0, 0)),
            pl.BlockSpec((1, HD), lambda i: (0, 0)),
        ],
        out_specs=pl.BlockSpec((NB, HD), lambda i: (i, 0)),
        out_shape=jax.ShapeDtypeStruct((N, HD), jnp.float32),
    )(z, stats, g.reshape(1, HD), b.reshape(1, HD))


def _tail_body(z_ref, s_ref, g_ref, b_ref, wf_ref, bf_ref, bat_ref, wo_ref,
               bo_ref, o_ref, acc_ref):
    i = pl.program_id(0)
    mu = s_ref[0:1, :] * (1.0 / N)
    var = s_ref[1:2, :] * (1.0 / N) - mu * mu
    scale = g_ref[...] * lax.rsqrt(var + 1e-5)
    h2 = jnp.maximum((z_ref[...] - mu) * scale + b_ref[...], 0.0)
    hf = jnp.maximum(
        jnp.dot(h2, wf_ref[...], preferred_element_type=jnp.float32) + bf_ref[...],
        0.0,
    )
    hf_ext = jnp.concatenate([hf, jnp.ones((NB, 1), jnp.float32)], axis=1)
    seg = lax.broadcasted_iota(jnp.int32, (NB, G), 1)
    onehot = (seg == bat_ref[...]).astype(jnp.float32)
    upd = lax.dot_general(onehot, hf_ext, (((0,), (0,)), ((), ())),
                          preferred_element_type=jnp.float32)

    @pl.when(i == 0)
    def _():
        acc_ref[...] = jnp.zeros_like(acc_ref)

    acc_ref[...] += upd

    @pl.when(i == N // NB - 1)
    def _():
        acc = acc_ref[...]
        pooled = acc[:, 0:HD] / jnp.maximum(acc[:, HD:HD + 1], 1.0)
        o_ref[...] = (
            jnp.dot(pooled, wo_ref[...], preferred_element_type=jnp.float32)
            + bo_ref[...]
        )


def _tail(z, stats, g, b, w_fin, b_fin, batch_row, w_out, b_out):
    c = w_out.shape[1]
    return pl.pallas_call(
        _tail_body,
        grid=(N // NB,),
        in_specs=[
            pl.BlockSpec((NB, HD), lambda i: (i, 0)),
            pl.BlockSpec((8, HD), lambda i: (0, 0)),
            pl.BlockSpec((1, HD), lambda i: (0, 0)),
            pl.BlockSpec((1, HD), lambda i: (0, 0)),
            pl.BlockSpec((HD, HD), lambda i: (0, 0)),
            pl.BlockSpec((1, HD), lambda i: (0, 0)),
            pl.BlockSpec((NB, 1), lambda i: (i, 0)),
            pl.BlockSpec((HD, c), lambda i: (0, 0)),
            pl.BlockSpec((1, c), lambda i: (0, 0)),
        ],
        out_specs=pl.BlockSpec((G, c), lambda i: (0, 0)),
        out_shape=jax.ShapeDtypeStruct((G, c), jnp.float32),
        scratch_shapes=[pltpu.VMEM((G, HD + 1), jnp.float32)],
    )(z, stats, g.reshape(1, HD), b.reshape(1, HD), w_fin,
      b_fin.reshape(1, HD), batch_row, w_out, b_out.reshape(1, c))


def _build_sc_msg(n, e, interpret=False):
    n_pad = -(-n // (NCORES * NSUB * 8)) * (NCORES * NSUB * 8)
    npc = n_pad // NCORES
    dummy = 512
    agg_rows = npc + dummy
    kc = 128
    nchunks = e // kc
    rows_per_tile = npc // NSUB
    full = nchunks // NSUB
    extra = nchunks - full * NSUB
    zrows = 64
    spans = []
    o = 0
    while o < rows_per_tile:
        sz = min(zrows, rows_per_tile - o)
        spans.append((o, sz))
        o += sz

    mesh = plsc.VectorSubcoreMesh(core_axis_name="c", subcore_axis_name="s",
                                  num_cores=NCORES, num_subcores=NSUB)

    @functools.partial(
        pl.kernel,
        out_type=jax.ShapeDtypeStruct((n_pad, HD), jnp.float32),
        mesh=mesh,
        scratch_types=[
            pltpu.VMEM_SHARED((agg_rows, HD), jnp.float32),
            [pltpu.VMEM((kc,), jnp.int32) for _ in range(3)],
            [pltpu.VMEM((kc,), jnp.int32) for _ in range(3)],
            [pltpu.VMEM((kc, HD), jnp.float32) for _ in range(3)],
            [pltpu.VMEM((kc * HD,), jnp.float32) for _ in range(3)],
            pltpu.VMEM((zrows, HD), jnp.float32),
            [pltpu.SemaphoreType.DMA for _ in range(3)],
            [pltpu.SemaphoreType.DMA for _ in range(3)],
            [pltpu.SemaphoreType.DMA for _ in range(3)],
        ],
        compiler_params=pltpu.CompilerParams(use_tc_tiling_on_sc=False),
        interpret=interpret,
    )
    def sc_msg(h_hbm, eaa_hbm, eab_hbm, src_hbm, dst_hbm, out_hbm,
               agg_sh, src_v, dst_v, rows_v, ea_v, zb,
               sem_ld, sem_g, sem_sc):
        c = lax.axis_index("c")
        s = lax.axis_index("s")
        base = c * npc
        row0 = s * rows_per_tile

        @plsc.parallel_loop(0, zrows, unroll=8)
        def _zz(j):
            zb[j, pl.ds(0, LANES)] = jnp.zeros((LANES,), jnp.float32)
            zb[j, pl.ds(LANES, LANES)] = jnp.zeros((LANES,), jnp.float32)

        for ofs, sz in spans:
            pltpu.sync_copy(zb.at[pl.ds(0, sz)],
                            agg_sh.at[pl.ds(row0 + ofs, sz)])
        plsc.subcore_barrier()

        nq = full + jnp.where(s < extra, 1, 0)

        half = kc * HD // 2

        def issue_ld(q, b):
            off = (q * NSUB + s) * kc
            po = off * (HD // 2)
            pltpu.async_copy(src_hbm.at[pl.ds(off, kc)], src_v[b], sem_ld[b])
            pltpu.async_copy(dst_hbm.at[pl.ds(off, kc)], dst_v[b], sem_ld[b])
            pltpu.async_copy(eaa_hbm.at[pl.ds(po, half)],
                             ea_v[b].at[pl.ds(0, half)], sem_ld[b])
            pltpu.async_copy(eab_hbm.at[pl.ds(po, half)],
                             ea_v[b].at[pl.ds(half, half)], sem_ld[b])

        def wait_ld(b):
            pltpu.make_async_copy(src_hbm.at[pl.ds(0, kc)], src_v[b],
                                  sem_ld[b]).wait()
            pltpu.make_async_copy(dst_hbm.at[pl.ds(0, kc)], dst_v[b],
                                  sem_ld[b]).wait()
            pltpu.make_async_copy(eaa_hbm.at[pl.ds(0, half)],
                                  ea_v[b].at[pl.ds(0, half)], sem_ld[b]).wait()
            pltpu.make_async_copy(eab_hbm.at[pl.ds(0, half)],
                                  ea_v[b].at[pl.ds(half, half)],
                                  sem_ld[b]).wait()

        def issue_gather(b):
            pltpu.async_copy(h_hbm.at[src_v[b]], rows_v[b], sem_g[b])

        def wait_gather(b):
            pltpu.make_async_copy(h_hbm.at[src_v[b]], rows_v[b],
                                  sem_g[b]).wait()

        def issue_scatter(b):
            pltpu.async_copy(rows_v[b], agg_sh.at[dst_v[b]], sem_sc[b],
                             add=True)

        def wait_scatter(b):
            pltpu.make_async_copy(rows_v[b], agg_sh.at[dst_v[b]],
                                  sem_sc[b]).wait()

        @pl.when(0 < nq)
        def _():
            issue_ld(0, 0)

        @pl.when(1 < nq)
        def _():
            issue_ld(1, 1)

        @pl.when(0 < nq)
        def _():
            wait_ld(0)
            issue_gather(0)

        @pl.loop(0, (nq + 2) // 3)
        def _macro(m):
            for j in range(3):
                q = m * 3 + j
                jn1 = (j + 1) % 3
                jn2 = (j + 2) % 3

                @pl.when(q + 2 < nq)
                def _(q=q, jn2=jn2):
                    @pl.when(q >= 1)
                    def _():
                        wait_scatter(jn2)

                    issue_ld(q + 2, jn2)

                @pl.when(q + 1 < nq)
                def _(q=q, jn1=jn1):
                    wait_ld(jn1)
                    issue_gather(jn1)

                @pl.when(q < nq)
                def _(q=q, j=j):
                    wait_gather(j)

                    @plsc.parallel_loop(0, kc, unroll=4)
                    def _rows(r):
                        a0 = rows_v[j][r, pl.ds(0, LANES)] + ea_v[j][pl.ds(r * HD, LANES)]
                        rows_v[j][r, pl.ds(0, LANES)] = jnp.maximum(a0, 0.0)
                        a1 = rows_v[j][r, pl.ds(LANES, LANES)] + ea_v[j][pl.ds(r * HD + LANES, LANES)]
                        rows_v[j][r, pl.ds(LANES, LANES)] = jnp.maximum(a1, 0.0)

                    @plsc.parallel_loop(0, kc // LANES, unroll=2)
                    def _didx(t):
                        d = dst_v[j][pl.ds(t * LANES, LANES)]
                        loc = d - base
                        oob = (loc < 0) | (loc >= npc)
                        dl = jnp.where(oob, npc + (d & (dummy - 1)), loc)
                        dst_v[j][pl.ds(t * LANES, LANES)] = dl

                    issue_scatter(j)

        for j in range(3):
            @pl.when(nq > j)
            def _(j=j):
                wait_scatter(j)

        plsc.subcore_barrier()

        for ofs, sz in spans:
            pltpu.sync_copy(agg_sh.at[pl.ds(row0 + ofs, sz)],
                            zb.at[pl.ds(0, sz)])
            pltpu.sync_copy(zb.at[pl.ds(0, sz)],
                            out_hbm.at[pl.ds(base + row0 + ofs, sz)])

    return sc_msg


@functools.lru_cache(maxsize=None)
def _sc_msg_cached():
    return _build_sc_msg(N, E)


def _sc_msg(h, ea_a, ea_b, src, dst):
    return _sc_msg_cached()(h, ea_a, ea_b, src, dst)[:N]


def kernel(x, edge_index, edge_attr, batch, w_node, b_node, w_edge, b_edge,
           w1_0, b1_0, w2_0, b2_0, bn_g_0, bn_b_0,
           w1_1, b1_1, w2_1, b2_1, bn_g_1, bn_b_1,
           w_fin, b_fin, w_out, b_out):
    perm = jnp.asarray(PERM128)
    src = edge_index[0].reshape(E // 128, 128)[:, perm].reshape(E)
    dst = edge_index[1].reshape(E // 128, 128)[:, perm].reshape(E)
    h0 = _linear(x, w_node, b_node, NB)
    af = edge_attr.reshape(E // 8, 8 * DE)
    eye_a = jnp.eye(8, 4, dtype=jnp.float32)
    eye_b = jnp.eye(8, 4, k=-4, dtype=jnp.float32)
    w8a = jnp.kron(eye_a, w_edge)
    w8b = jnp.kron(eye_b, w_edge)
    b128 = jnp.tile(b_edge, 4)
    oa, ob = _edge_linear_packed(af, w8a, w8b, b128)
    ea_a = oa.reshape(E * HD // 2)
    ea_b = ob.reshape(E * HD // 2)
    agg0 = _sc_msg(h0, ea_a, ea_b, src, dst)
    z0, s0 = _mlp_stats(h0, agg0, w1_0, b1_0, w2_0, b2_0)
    h1 = _norm_relu(z0, s0, bn_g_0, bn_b_0)
    agg1 = _sc_msg(h1, ea_a, ea_b, src, dst)
    z1, s1 = _mlp_stats(h1, agg1, w1_1, b1_1, w2_1, b2_1)
    return _tail(z1, s1, bn_g_1, bn_b_1, w_fin, b_fin,
                 batch.reshape(N, 1), w_out, b_out)
